# Initial kernel scaffold; baseline (speedup 1.0000x reference)
import jax
import jax.numpy as jnp
from jax.experimental import pallas as pl


def kernel(x, edge_index, batch, conv_W0, conv_b0, conv_W1, conv_b1, conv_W2, conv_b2, ln_s0, ln_b0, ln_s1, ln_b1, ln_s2, ln_b2, gh_W1, gh_b1, gh_W2, gh_b2, gh_W3, gh_b3):
    raise NotImplementedError("write your pallas kernel here")


# speedup vs baseline: 7.0029x; 7.0029x over previous
import functools

import jax
import jax.numpy as jnp
from jax import lax
from jax.experimental import pallas as pl
from jax.experimental.pallas import tpu as pltpu
from jax.experimental.pallas import tpu_sc as plsc

N = 10000
NP = 10240
E = 160000
H = 256
HF = 128
NB = 64
OUT = 12
NC, NS = 2, 16
TN = 1024
GI = NP // TN
EPT = E // NS
KCH = 80
NCHUNK = EPT // KCH
RPT = NP // NS
DW = 16


def _deg_body(dst_hbm, out_hbm, didx, ones_v, acc):
    c = lax.axis_index("c")
    s = lax.axis_index("s")

    @pl.when(c == 0)
    def _():
        zv = jnp.zeros((16,), jnp.float32)
        for j in range(KCH // 16):
            ones_v[pl.ds(16 * j, 16)] = zv
        for t in range(RPT // KCH):
            pltpu.sync_copy(ones_v, acc.at[pl.ds(s * RPT + t * KCH, KCH)])
        ov = jnp.ones((16,), jnp.float32)
        for j in range(KCH // 16):
            ones_v[pl.ds(16 * j, 16)] = ov
        plsc.subcore_barrier()

        def _chunk(k, carry):
            base = s * EPT + k * KCH
            pltpu.sync_copy(dst_hbm.at[pl.ds(base, KCH)], didx)
            pltpu.sync_copy(ones_v, acc.at[didx], add=True)
            return carry

        lax.fori_loop(0, NCHUNK, _chunk, 0)
        plsc.subcore_barrier()
        pltpu.sync_copy(acc.at[pl.ds(s * RPT, RPT)],
                        out_hbm.at[pl.ds(s * RPT, RPT)])


def _edge_body(u_hbm, src_hbm, dst_hbm, out_hbm, sidx, didx, rows, acc, sem):
    c = lax.axis_index("c")
    s = lax.axis_index("s")
    zv = jnp.zeros((16,), jnp.float32)

    def _zrow(r, carry):
        for j in range(HF // 16):
            rows[r, pl.ds(16 * j, 16)] = zv
        return carry

    lax.fori_loop(0, KCH, _zrow, 0)
    for t in range(RPT // KCH):
        pltpu.sync_copy(rows, acc.at[pl.ds(s * RPT + t * KCH, KCH)])
    plsc.subcore_barrier()

    off = c * NP

    def _chunk(k, carry):
        base = s * EPT + k * KCH
        pltpu.sync_copy(src_hbm.at[pl.ds(base, KCH)], sidx)
        pltpu.sync_copy(dst_hbm.at[pl.ds(base, KCH)], didx)
        for j in range(KCH // 16):
            sl = pl.ds(16 * j, 16)
            sidx[sl] = sidx[sl] + off
        pltpu.async_copy(u_hbm.at[sidx], rows, sem).wait()
        pltpu.sync_copy(rows, acc.at[didx], add=True)
        return carry

    lax.fori_loop(0, NCHUNK, _chunk, 0)
    plsc.subcore_barrier()
    pltpu.sync_copy(acc.at[pl.ds(s * RPT, RPT)],
                    out_hbm.at[pl.ds(c * NP + s * RPT, RPT)])


@functools.cache
def _sc_kernels():
    mesh = plsc.VectorSubcoreMesh(
        core_axis_name="c", subcore_axis_name="s",
        num_cores=NC, num_subcores=NS)
    deg = pl.kernel(
        _deg_body,
        out_type=jax.ShapeDtypeStruct((NP,), jnp.float32),
        mesh=mesh,
        scratch_types=[
            pltpu.VMEM((KCH,), jnp.int32),
            pltpu.VMEM((KCH,), jnp.float32),
            pltpu.VMEM_SHARED((NP,), jnp.float32),
        ],
        name="sc_degree",
    )
    edge = pl.kernel(
        _edge_body,
        out_type=jax.ShapeDtypeStruct((NC * NP, HF), jnp.float32),
        mesh=mesh,
        scratch_types=[
            pltpu.VMEM((KCH,), jnp.int32),
            pltpu.VMEM((KCH,), jnp.int32),
            pltpu.VMEM((KCH, HF), jnp.float32),
            pltpu.VMEM_SHARED((NP, HF), jnp.float32),
            pltpu.SemaphoreType.DMA,
        ],
        name="sc_edge_pass",
    )
    return deg, edge


def _deg_sc(dst):
    return _sc_kernels()[0](dst)


def _edge_sc(u, src, dst):
    return _sc_kernels()[1](u, src, dst)


def _dinv_of(deg_ref):
    return lax.rsqrt(deg_ref[:, :1] + 1.0)


def _pre0_body(deg_ref, x_ref, w_ref, u_ref):
    dinv = _dinv_of(deg_ref)
    u_ref[...] = jnp.dot(x_ref[...], w_ref[...],
                         preferred_element_type=jnp.float32) * dinv


def _norm_act(deg_ref, sa, sb, ua, ub, b_ref, lns_ref, lnb_ref):
    dinv = _dinv_of(deg_ref)
    t = jnp.concatenate([sa[...] + ua[...], sb[...] + ub[...]], axis=1)
    t = t * dinv + b_ref[...]
    mu = jnp.mean(t, axis=1, keepdims=True)
    var = jnp.mean((t - mu) ** 2, axis=1, keepdims=True)
    y = (t - mu) * lax.rsqrt(var + 1e-5) * lns_ref[...] + lnb_ref[...]
    return jnp.maximum(y, 0.0), dinv


def _mid_body_nores(deg_ref, sa, sb, ua, ub, b_ref, lns_ref, lnb_ref, w_ref,
                    h_out, u_out):
    h, dinv = _norm_act(deg_ref, sa, sb, ua, ub, b_ref, lns_ref, lnb_ref)
    h_out[...] = h
    u_out[...] = jnp.dot(h, w_ref[...],
                         preferred_element_type=jnp.float32) * dinv


def _mid_body_res(deg_ref, sa, sb, ua, ub, b_ref, lns_ref, lnb_ref, hp_ref,
                  w_ref, h_out, u_out):
    h, dinv = _norm_act(deg_ref, sa, sb, ua, ub, b_ref, lns_ref, lnb_ref)
    h = h + hp_ref[...]
    h_out[...] = h
    u_out[...] = jnp.dot(h, w_ref[...],
                         preferred_element_type=jnp.float32) * dinv


def _fin_body(deg_ref, sa, sb, ua, ub, b_ref, lns_ref, lnb_ref, hp_ref,
              batch_ref, w1, b1, w2, b2, w3, b3, out_ref, acc, cnt):
    i = pl.program_id(0)

    @pl.when(i == 0)
    def _():
        acc[...] = jnp.zeros_like(acc)
        cnt[...] = jnp.zeros_like(cnt)

    h, _ = _norm_act(deg_ref, sa, sb, ua, ub, b_ref, lns_ref, lnb_ref)
    h = h + hp_ref[...]
    seg = batch_ref[...]
    mask = (lax.broadcasted_iota(jnp.int32, (TN, NB), 1) == seg
            ).astype(jnp.float32)
    acc[...] += lax.dot_general(mask, h, (((0,), (0,)), ((), ())),
                                preferred_element_type=jnp.float32)
    cnt[...] += lax.dot_general(mask, jnp.ones((TN, 1), jnp.float32),
                                (((0,), (0,)), ((), ())),
                                preferred_element_type=jnp.float32)

    @pl.when(i == GI - 1)
    def _():
        pooled = acc[...] / jnp.maximum(cnt[...], 1.0)
        g = jnp.maximum(jnp.dot(pooled, w1[...],
                                preferred_element_type=jnp.float32)
                        + b1[...], 0.0)
        g = jnp.maximum(jnp.dot(g, w2[...],
                                preferred_element_type=jnp.float32)
                        + b2[...], 0.0)
        out_ref[...] = jnp.dot(g, w3[...],
                               preferred_element_type=jnp.float32) + b3[...]


def _node_spec(shape=(TN, H)):
    return pl.BlockSpec(shape, lambda i, c: (i, 0))


def _half_spec():
    return pl.BlockSpec((TN, HF), lambda i, c: (c * GI + i, 0))


def _halves_specs():
    a = pl.BlockSpec((TN, HF), lambda i: (i, 0))
    b = pl.BlockSpec((TN, HF), lambda i: (GI + i, 0))
    return a, b


_pre0 = pl.pallas_call(
    _pre0_body,
    grid=(GI, NC),
    in_specs=[
        pl.BlockSpec((TN, 1), lambda i, c: (i, 0)),
        pl.BlockSpec((TN, H), lambda i, c: (i, 0)),
        pl.BlockSpec((H, HF), lambda i, c: (0, c)),
    ],
    out_specs=_half_spec(),
    out_shape=jax.ShapeDtypeStruct((NC * NP, HF), jnp.float32),
    name="tc_pre0",
)


def _make_mid(residual):
    body = _mid_body_res if residual else _mid_body_nores
    sa2 = pl.BlockSpec((TN, HF), lambda i, c: (i, 0))
    sb2 = pl.BlockSpec((TN, HF), lambda i, c: (GI + i, 0))
    row = pl.BlockSpec((1, H), lambda i, c: (0, 0))
    specs = [
        pl.BlockSpec((TN, 1), lambda i, c: (i, 0)),
        sa2, sb2,
        pl.BlockSpec((TN, HF), lambda i, c: (i, 0)),
        pl.BlockSpec((TN, HF), lambda i, c: (GI + i, 0)),
        row, row, row,
    ]
    if residual:
        specs.append(pl.BlockSpec((TN, H), lambda i, c: (i, 0)))
    specs.append(pl.BlockSpec((H, HF), lambda i, c: (0, c)))
    return pl.pallas_call(
        body,
        grid=(GI, NC),
        in_specs=specs,
        out_specs=[
            pl.BlockSpec((TN, H), lambda i, c: (i, 0)),
            _half_spec(),
        ],
        out_shape=[
            jax.ShapeDtypeStruct((NP, H), jnp.float32),
            jax.ShapeDtypeStruct((NC * NP, HF), jnp.float32),
        ],
        name="tc_mid_res" if residual else "tc_mid",
    )


_mid0 = _make_mid(False)
_mid1 = _make_mid(True)

_sa1, _sb1 = _halves_specs()
_row1 = pl.BlockSpec((1, H), lambda i: (0, 0))
_fin = pl.pallas_call(
    _fin_body,
    grid=(GI,),
    in_specs=[
        pl.BlockSpec((TN, 1), lambda i: (i, 0)),
        _sa1, _sb1,
        pl.BlockSpec((TN, HF), lambda i: (i, 0)),
        pl.BlockSpec((TN, HF), lambda i: (GI + i, 0)),
        _row1, _row1, _row1,
        pl.BlockSpec((TN, H), lambda i: (i, 0)),
        pl.BlockSpec((TN, 1), lambda i: (i, 0)),
        pl.BlockSpec((H, H), lambda i: (0, 0)),
        pl.BlockSpec((1, H), lambda i: (0, 0)),
        pl.BlockSpec((H, HF), lambda i: (0, 0)),
        pl.BlockSpec((1, HF), lambda i: (0, 0)),
        pl.BlockSpec((HF, OUT), lambda i: (0, 0)),
        pl.BlockSpec((1, OUT), lambda i: (0, 0)),
    ],
    out_specs=pl.BlockSpec((NB, OUT), lambda i: (0, 0)),
    out_shape=jax.ShapeDtypeStruct((NB, OUT), jnp.float32),
    scratch_shapes=[
        pltpu.VMEM((NB, H), jnp.float32),
        pltpu.VMEM((NB, 1), jnp.float32),
    ],
    name="tc_final",
)


def kernel(x, edge_index, batch,
           conv_W0, conv_b0, conv_W1, conv_b1, conv_W2, conv_b2,
           ln_s0, ln_b0, ln_s1, ln_b1, ln_s2, ln_b2,
           gh_W1, gh_b1, gh_W2, gh_b2, gh_W3, gh_b3):
    src = edge_index[0]
    dst = edge_index[1]
    xp = jnp.pad(x, ((0, NP - N), (0, 0)))
    batch_p = jnp.pad(batch, (0, NP - N), constant_values=NB).reshape(NP, 1)

    deg = _deg_sc(dst).reshape(NP, 1)
    u0 = _pre0(deg, xp, conv_W0)
    s0 = _edge_sc(u0, src, dst)
    h0, u1 = _mid0(deg, s0, s0, u0, u0,
                   conv_b0.reshape(1, H), ln_s0.reshape(1, H),
                   ln_b0.reshape(1, H), conv_W1)
    s1 = _edge_sc(u1, src, dst)
    h1, u2 = _mid1(deg, s1, s1, u1, u1,
                   conv_b1.reshape(1, H), ln_s1.reshape(1, H),
                   ln_b1.reshape(1, H), h0, conv_W2)
    s2 = _edge_sc(u2, src, dst)
    out = _fin(deg, s2, s2, u2, u2,
               conv_b2.reshape(1, H), ln_s2.reshape(1, H),
               ln_b2.reshape(1, H), h1, batch_p,
               gh_W1, gh_b1.reshape(1, H), gh_W2, gh_b2.reshape(1, HF),
               gh_W3, gh_b3.reshape(1, OUT))
    return out


# speedup vs baseline: 13.9848x; 1.9970x over previous
import functools

import jax
import jax.numpy as jnp
from jax import lax
from jax.experimental import pallas as pl
from jax.experimental.pallas import tpu as pltpu
from jax.experimental.pallas import tpu_sc as plsc

N = 10000
NP = 10240
E = 160000
H = 256
HF = 128
NB = 64
OUT = 12
NC, NS = 2, 16
TN = 1024
GI = NP // TN
EPT = E // NS
KCH = 80
NCHUNK = EPT // KCH
RPT = NP // NS
DW = 16


def _deg_body(dst_hbm, out_hbm, didx, ones_v, acc):
    c = lax.axis_index("c")
    s = lax.axis_index("s")

    @pl.when(c == 0)
    def _():
        zv = jnp.zeros((16,), jnp.float32)
        for j in range(KCH // 16):
            ones_v[pl.ds(16 * j, 16)] = zv
        for t in range(RPT // KCH):
            pltpu.sync_copy(ones_v, acc.at[pl.ds(s * RPT + t * KCH, KCH)])
        ov = jnp.ones((16,), jnp.float32)
        for j in range(KCH // 16):
            ones_v[pl.ds(16 * j, 16)] = ov
        plsc.subcore_barrier()

        def _chunk(k, carry):
            base = s * EPT + k * KCH
            pltpu.sync_copy(dst_hbm.at[pl.ds(base, KCH)], didx)
            pltpu.sync_copy(ones_v, acc.at[didx], add=True)
            return carry

        lax.fori_loop(0, NCHUNK, _chunk, 0)
        plsc.subcore_barrier()
        pltpu.sync_copy(acc.at[pl.ds(s * RPT, RPT)],
                        out_hbm.at[pl.ds(s * RPT, RPT)])


def _edge_body(u_hbm, src_hbm, dst_hbm, out_hbm, sidx, dA, dB, rowsA, rowsB,
               acc, semA, semB, semDA, semDB):
    c = lax.axis_index("c")
    s = lax.axis_index("s")
    zv = jnp.zeros((16,), jnp.float32)

    def _zrow(r, carry):
        for j in range(HF // 16):
            rowsA[r, pl.ds(16 * j, 16)] = zv
        return carry

    lax.fori_loop(0, KCH, _zrow, 0)
    for t in range(RPT // KCH):
        pltpu.sync_copy(rowsA, acc.at[pl.ds(s * RPT + t * KCH, KCH)])

    base = s * EPT
    pltpu.sync_copy(src_hbm.at[pl.ds(base, EPT)], sidx)
    off = c * NP

    def _off(g, carry):
        sl = pl.ds(16 * g, 16)
        sidx[sl] = sidx[sl] + off
        return carry

    lax.fori_loop(0, EPT // 16, _off, 0)
    plsc.subcore_barrier()

    bufs = ((rowsA, dA, semA, semDA), (rowsB, dB, semB, semDB))

    def _issue(k, parity):
        rows, d, semG, semD = bufs[parity]
        pltpu.async_copy(dst_hbm.at[pl.ds(base + k * KCH, KCH)], d, semD)
        pltpu.async_copy(u_hbm.at[sidx.at[pl.ds(k * KCH, KCH)]], rows, semG)

    def _wait_scatter(parity):
        rows, d, semG, semD = bufs[parity]
        pltpu.make_async_copy(dst_hbm.at[pl.ds(0, KCH)], d, semD).wait()
        pltpu.make_async_copy(u_hbm.at[pl.ds(0, KCH)], rows, semG).wait()
        pltpu.sync_copy(rows, acc.at[d], add=True)

    _issue(0, 0)

    def _pair(i, carry):
        for p in range(2):
            k = 2 * i + p
            _issue(k + 1, 1 - p)
            _wait_scatter(p)
        return carry

    lax.fori_loop(0, NCHUNK // 2, _pair, 0)
    _wait_scatter(0)

    plsc.subcore_barrier()
    pltpu.sync_copy(acc.at[pl.ds(s * RPT, RPT)],
                    out_hbm.at[pl.ds(c * NP + s * RPT, RPT)])


@functools.cache
def _sc_kernels():
    mesh = plsc.VectorSubcoreMesh(
        core_axis_name="c", subcore_axis_name="s",
        num_cores=NC, num_subcores=NS)
    deg = pl.kernel(
        _deg_body,
        out_type=jax.ShapeDtypeStruct((NP,), jnp.float32),
        mesh=mesh,
        scratch_types=[
            pltpu.VMEM((KCH,), jnp.int32),
            pltpu.VMEM((KCH,), jnp.float32),
            pltpu.VMEM_SHARED((NP,), jnp.float32),
        ],
        name="sc_degree",
    )
    edge = pl.kernel(
        _edge_body,
        out_type=jax.ShapeDtypeStruct((NC * NP, HF), jnp.float32),
        mesh=mesh,
        scratch_types=[
            pltpu.VMEM((EPT,), jnp.int32),
            pltpu.VMEM((KCH,), jnp.int32),
            pltpu.VMEM((KCH,), jnp.int32),
            pltpu.VMEM((KCH, HF), jnp.float32),
            pltpu.VMEM((KCH, HF), jnp.float32),
            pltpu.VMEM_SHARED((NP, HF), jnp.float32),
            pltpu.SemaphoreType.DMA,
            pltpu.SemaphoreType.DMA,
            pltpu.SemaphoreType.DMA,
            pltpu.SemaphoreType.DMA,
        ],
        name="sc_edge_pass",
    )
    return deg, edge


def _deg_sc(dst):
    return _sc_kernels()[0](dst)


def _edge_sc(u, src, dst):
    return _sc_kernels()[1](u, src, dst)


def _dinv_of(deg_ref):
    return lax.rsqrt(deg_ref[:, :1] + 1.0)


def _pre0_body(deg_ref, x_ref, w_ref, u_ref):
    dinv = _dinv_of(deg_ref)
    u_ref[...] = jnp.dot(x_ref[...], w_ref[...],
                         preferred_element_type=jnp.float32) * dinv


def _norm_act(deg_ref, sa, sb, ua, ub, b_ref, lns_ref, lnb_ref):
    dinv = _dinv_of(deg_ref)
    t = jnp.concatenate([sa[...] + ua[...], sb[...] + ub[...]], axis=1)
    t = t * dinv + b_ref[...]
    mu = jnp.mean(t, axis=1, keepdims=True)
    var = jnp.mean((t - mu) ** 2, axis=1, keepdims=True)
    y = (t - mu) * lax.rsqrt(var + 1e-5) * lns_ref[...] + lnb_ref[...]
    return jnp.maximum(y, 0.0), dinv


def _mid_body_nores(deg_ref, sa, sb, ua, ub, b_ref, lns_ref, lnb_ref, w_ref,
                    h_out, u_out):
    h, dinv = _norm_act(deg_ref, sa, sb, ua, ub, b_ref, lns_ref, lnb_ref)
    h_out[...] = h
    u_out[...] = jnp.dot(h, w_ref[...],
                         preferred_element_type=jnp.float32) * dinv


def _mid_body_res(deg_ref, sa, sb, ua, ub, b_ref, lns_ref, lnb_ref, hp_ref,
                  w_ref, h_out, u_out):
    h, dinv = _norm_act(deg_ref, sa, sb, ua, ub, b_ref, lns_ref, lnb_ref)
    h = h + hp_ref[...]
    h_out[...] = h
    u_out[...] = jnp.dot(h, w_ref[...],
                         preferred_element_type=jnp.float32) * dinv


def _fin_body(deg_ref, sa, sb, ua, ub, b_ref, lns_ref, lnb_ref, hp_ref,
              batch_ref, w1, b1, w2, b2, w3, b3, out_ref, acc, cnt):
    i = pl.program_id(0)

    @pl.when(i == 0)
    def _():
        acc[...] = jnp.zeros_like(acc)
        cnt[...] = jnp.zeros_like(cnt)

    h, _ = _norm_act(deg_ref, sa, sb, ua, ub, b_ref, lns_ref, lnb_ref)
    h = h + hp_ref[...]
    seg = batch_ref[...]
    mask = (lax.broadcasted_iota(jnp.int32, (TN, NB), 1) == seg
            ).astype(jnp.float32)
    acc[...] += lax.dot_general(mask, h, (((0,), (0,)), ((), ())),
                                preferred_element_type=jnp.float32)
    cnt[...] += lax.dot_general(mask, jnp.ones((TN, 1), jnp.float32),
                                (((0,), (0,)), ((), ())),
                                preferred_element_type=jnp.float32)

    @pl.when(i == GI - 1)
    def _():
        pooled = acc[...] / jnp.maximum(cnt[...], 1.0)
        g = jnp.maximum(jnp.dot(pooled, w1[...],
                                preferred_element_type=jnp.float32)
                        + b1[...], 0.0)
        g = jnp.maximum(jnp.dot(g, w2[...],
                                preferred_element_type=jnp.float32)
                        + b2[...], 0.0)
        out_ref[...] = jnp.dot(g, w3[...],
                               preferred_element_type=jnp.float32) + b3[...]


def _node_spec(shape=(TN, H)):
    return pl.BlockSpec(shape, lambda i, c: (i, 0))


def _half_spec():
    return pl.BlockSpec((TN, HF), lambda i, c: (c * GI + i, 0))


def _halves_specs():
    a = pl.BlockSpec((TN, HF), lambda i: (i, 0))
    b = pl.BlockSpec((TN, HF), lambda i: (GI + i, 0))
    return a, b


_pre0 = pl.pallas_call(
    _pre0_body,
    grid=(GI, NC),
    in_specs=[
        pl.BlockSpec((TN, 1), lambda i, c: (i, 0)),
        pl.BlockSpec((TN, H), lambda i, c: (i, 0)),
        pl.BlockSpec((H, HF), lambda i, c: (0, c)),
    ],
    out_specs=_half_spec(),
    out_shape=jax.ShapeDtypeStruct((NC * NP, HF), jnp.float32),
    name="tc_pre0",
)


def _make_mid(residual):
    body = _mid_body_res if residual else _mid_body_nores
    sa2 = pl.BlockSpec((TN, HF), lambda i, c: (i, 0))
    sb2 = pl.BlockSpec((TN, HF), lambda i, c: (GI + i, 0))
    row = pl.BlockSpec((1, H), lambda i, c: (0, 0))
    specs = [
        pl.BlockSpec((TN, 1), lambda i, c: (i, 0)),
        sa2, sb2,
        pl.BlockSpec((TN, HF), lambda i, c: (i, 0)),
        pl.BlockSpec((TN, HF), lambda i, c: (GI + i, 0)),
        row, row, row,
    ]
    if residual:
        specs.append(pl.BlockSpec((TN, H), lambda i, c: (i, 0)))
    specs.append(pl.BlockSpec((H, HF), lambda i, c: (0, c)))
    return pl.pallas_call(
        body,
        grid=(GI, NC),
        in_specs=specs,
        out_specs=[
            pl.BlockSpec((TN, H), lambda i, c: (i, 0)),
            _half_spec(),
        ],
        out_shape=[
            jax.ShapeDtypeStruct((NP, H), jnp.float32),
            jax.ShapeDtypeStruct((NC * NP, HF), jnp.float32),
        ],
        name="tc_mid_res" if residual else "tc_mid",
    )


_mid0 = _make_mid(False)
_mid1 = _make_mid(True)

_sa1, _sb1 = _halves_specs()
_row1 = pl.BlockSpec((1, H), lambda i: (0, 0))
_fin = pl.pallas_call(
    _fin_body,
    grid=(GI,),
    in_specs=[
        pl.BlockSpec((TN, 1), lambda i: (i, 0)),
        _sa1, _sb1,
        pl.BlockSpec((TN, HF), lambda i: (i, 0)),
        pl.BlockSpec((TN, HF), lambda i: (GI + i, 0)),
        _row1, _row1, _row1,
        pl.BlockSpec((TN, H), lambda i: (i, 0)),
        pl.BlockSpec((TN, 1), lambda i: (i, 0)),
        pl.BlockSpec((H, H), lambda i: (0, 0)),
        pl.BlockSpec((1, H), lambda i: (0, 0)),
        pl.BlockSpec((H, HF), lambda i: (0, 0)),
        pl.BlockSpec((1, HF), lambda i: (0, 0)),
        pl.BlockSpec((HF, OUT), lambda i: (0, 0)),
        pl.BlockSpec((1, OUT), lambda i: (0, 0)),
    ],
    out_specs=pl.BlockSpec((NB, OUT), lambda i: (0, 0)),
    out_shape=jax.ShapeDtypeStruct((NB, OUT), jnp.float32),
    scratch_shapes=[
        pltpu.VMEM((NB, H), jnp.float32),
        pltpu.VMEM((NB, 1), jnp.float32),
    ],
    name="tc_final",
)


def kernel(x, edge_index, batch,
           conv_W0, conv_b0, conv_W1, conv_b1, conv_W2, conv_b2,
           ln_s0, ln_b0, ln_s1, ln_b1, ln_s2, ln_b2,
           gh_W1, gh_b1, gh_W2, gh_b2, gh_W3, gh_b3):
    src = edge_index[0]
    dst = edge_index[1]
    xp = jnp.pad(x, ((0, NP - N), (0, 0)))
    batch_p = jnp.pad(batch, (0, NP - N), constant_values=NB).reshape(NP, 1)

    deg = _deg_sc(dst).reshape(NP, 1)
    u0 = _pre0(deg, xp, conv_W0)
    s0 = _edge_sc(u0, src, dst)
    h0, u1 = _mid0(deg, s0, s0, u0, u0,
                   conv_b0.reshape(1, H), ln_s0.reshape(1, H),
                   ln_b0.reshape(1, H), conv_W1)
    s1 = _edge_sc(u1, src, dst)
    h1, u2 = _mid1(deg, s1, s1, u1, u1,
                   conv_b1.reshape(1, H), ln_s1.reshape(1, H),
                   ln_b1.reshape(1, H), h0, conv_W2)
    s2 = _edge_sc(u2, src, dst)
    out = _fin(deg, s2, s2, u2, u2,
               conv_b2.reshape(1, H), ln_s2.reshape(1, H),
               ln_b2.reshape(1, H), h1, batch_p,
               gh_W1, gh_b1.reshape(1, H), gh_W2, gh_b2.reshape(1, HF),
               gh_W3, gh_b3.reshape(1, OUT))
    return out


# speedup vs baseline: 14.7833x; 1.0571x over previous
import functools

import jax
import jax.numpy as jnp
from jax import lax
from jax.experimental import pallas as pl
from jax.experimental.pallas import tpu as pltpu
from jax.experimental.pallas import tpu_sc as plsc

N = 10000
NP = 10240
E = 160000
H = 256
HF = 128
NB = 64
OUT = 12
NC, NS = 2, 16
TN = 1024
GI = NP // TN
EPT = E // NS
KCH = 80
NCHUNK = EPT // KCH
RPT = NP // NS
DW = 16


def _deg_body(dst_hbm, out_hbm, dA, dB, ones_v, acc, semDA, semDB):
    c = lax.axis_index("c")
    s = lax.axis_index("s")

    @pl.when(c == 0)
    def _():
        zv = jnp.zeros((16,), jnp.float32)
        for j in range(KCH // 16):
            ones_v[pl.ds(16 * j, 16)] = zv
        for t in range(RPT // KCH):
            pltpu.sync_copy(ones_v, acc.at[pl.ds(s * RPT + t * KCH, KCH)])
        ov = jnp.ones((16,), jnp.float32)
        for j in range(KCH // 16):
            ones_v[pl.ds(16 * j, 16)] = ov
        plsc.subcore_barrier()

        base = s * EPT
        bufs = ((dA, semDA), (dB, semDB))

        def _issue(k, parity):
            d, semD = bufs[parity]
            pltpu.async_copy(dst_hbm.at[pl.ds(base + k * KCH, KCH)], d, semD)

        def _wait_scatter(parity):
            d, semD = bufs[parity]
            pltpu.make_async_copy(dst_hbm.at[pl.ds(0, KCH)], d, semD).wait()
            pltpu.sync_copy(ones_v, acc.at[d], add=True)

        _issue(0, 0)

        def _pair(i, carry):
            for p in range(2):
                k = 2 * i + p
                _issue(k + 1, 1 - p)
                _wait_scatter(p)
            return carry

        lax.fori_loop(0, NCHUNK // 2, _pair, 0)
        _wait_scatter(0)
        plsc.subcore_barrier()
        pltpu.sync_copy(acc.at[pl.ds(s * RPT, RPT)],
                        out_hbm.at[pl.ds(s * RPT, RPT)])


def _edge_body(u_hbm, src_hbm, dst_hbm, out_hbm, sidx, dA, dB, rowsA, rowsB,
               acc, semA, semB, semDA, semDB):
    c = lax.axis_index("c")
    s = lax.axis_index("s")
    zv = jnp.zeros((16,), jnp.float32)

    def _zrow(r, carry):
        for j in range(HF // 16):
            rowsA[r, pl.ds(16 * j, 16)] = zv
        return carry

    lax.fori_loop(0, KCH, _zrow, 0)
    for t in range(RPT // KCH):
        pltpu.sync_copy(rowsA, acc.at[pl.ds(s * RPT + t * KCH, KCH)])

    base = s * EPT
    pltpu.sync_copy(src_hbm.at[pl.ds(base, EPT)], sidx)
    off = c * NP

    def _off(g, carry):
        sl = pl.ds(16 * g, 16)
        sidx[sl] = sidx[sl] + off
        return carry

    lax.fori_loop(0, EPT // 16, _off, 0)
    plsc.subcore_barrier()

    bufs = ((rowsA, dA, semA, semDA), (rowsB, dB, semB, semDB))

    def _issue(k, parity):
        rows, d, semG, semD = bufs[parity]
        pltpu.async_copy(dst_hbm.at[pl.ds(base + k * KCH, KCH)], d, semD)
        pltpu.async_copy(u_hbm.at[sidx.at[pl.ds(k * KCH, KCH)]], rows, semG)

    def _wait_scatter(parity):
        rows, d, semG, semD = bufs[parity]
        pltpu.make_async_copy(dst_hbm.at[pl.ds(0, KCH)], d, semD).wait()
        pltpu.make_async_copy(u_hbm.at[pl.ds(0, KCH)], rows, semG).wait()
        pltpu.sync_copy(rows, acc.at[d], add=True)

    _issue(0, 0)

    def _pair(i, carry):
        for p in range(2):
            k = 2 * i + p
            _issue(k + 1, 1 - p)
            _wait_scatter(p)
        return carry

    lax.fori_loop(0, NCHUNK // 2, _pair, 0)
    _wait_scatter(0)

    plsc.subcore_barrier()
    pltpu.sync_copy(acc.at[pl.ds(s * RPT, RPT)],
                    out_hbm.at[pl.ds(c * NP + s * RPT, RPT)])


@functools.cache
def _sc_kernels():
    mesh = plsc.VectorSubcoreMesh(
        core_axis_name="c", subcore_axis_name="s",
        num_cores=NC, num_subcores=NS)
    deg = pl.kernel(
        _deg_body,
        out_type=jax.ShapeDtypeStruct((NP,), jnp.float32),
        mesh=mesh,
        scratch_types=[
            pltpu.VMEM((KCH,), jnp.int32),
            pltpu.VMEM((KCH,), jnp.int32),
            pltpu.VMEM((KCH,), jnp.float32),
            pltpu.VMEM_SHARED((NP,), jnp.float32),
            pltpu.SemaphoreType.DMA,
            pltpu.SemaphoreType.DMA,
        ],
        name="sc_degree",
    )
    edge = pl.kernel(
        _edge_body,
        out_type=jax.ShapeDtypeStruct((NC * NP, HF), jnp.float32),
        mesh=mesh,
        scratch_types=[
            pltpu.VMEM((EPT,), jnp.int32),
            pltpu.VMEM((KCH,), jnp.int32),
            pltpu.VMEM((KCH,), jnp.int32),
            pltpu.VMEM((KCH, HF), jnp.float32),
            pltpu.VMEM((KCH, HF), jnp.float32),
            pltpu.VMEM_SHARED((NP, HF), jnp.float32),
            pltpu.SemaphoreType.DMA,
            pltpu.SemaphoreType.DMA,
            pltpu.SemaphoreType.DMA,
            pltpu.SemaphoreType.DMA,
        ],
        name="sc_edge_pass",
    )
    return deg, edge


def _deg_sc(dst):
    return _sc_kernels()[0](dst)


def _edge_sc(u, src, dst):
    return _sc_kernels()[1](u, src, dst)


def _dinv_of(deg_ref):
    return lax.rsqrt(deg_ref[:, :1] + 1.0)


def _pre0_body(deg_ref, x_ref, w_ref, u_ref):
    dinv = _dinv_of(deg_ref)
    u_ref[...] = jnp.dot(x_ref[...], w_ref[...],
                         preferred_element_type=jnp.float32) * dinv


def _norm_act(deg_ref, sa, sb, ua, ub, b_ref, lns_ref, lnb_ref):
    dinv = _dinv_of(deg_ref)
    t = jnp.concatenate([sa[...] + ua[...], sb[...] + ub[...]], axis=1)
    t = t * dinv + b_ref[...]
    mu = jnp.mean(t, axis=1, keepdims=True)
    var = jnp.mean((t - mu) ** 2, axis=1, keepdims=True)
    y = (t - mu) * lax.rsqrt(var + 1e-5) * lns_ref[...] + lnb_ref[...]
    return jnp.maximum(y, 0.0), dinv


def _mid_body_nores(deg_ref, sa, sb, ua, ub, b_ref, lns_ref, lnb_ref, w_ref,
                    h_out, u_out):
    h, dinv = _norm_act(deg_ref, sa, sb, ua, ub, b_ref, lns_ref, lnb_ref)
    h_out[...] = h
    u_out[...] = jnp.dot(h, w_ref[...],
                         preferred_element_type=jnp.float32) * dinv


def _mid_body_res(deg_ref, sa, sb, ua, ub, b_ref, lns_ref, lnb_ref, hp_ref,
                  w_ref, h_out, u_out):
    h, dinv = _norm_act(deg_ref, sa, sb, ua, ub, b_ref, lns_ref, lnb_ref)
    h = h + hp_ref[...]
    h_out[...] = h
    u_out[...] = jnp.dot(h, w_ref[...],
                         preferred_element_type=jnp.float32) * dinv


def _fin_body(deg_ref, sa, sb, ua, ub, b_ref, lns_ref, lnb_ref, hp_ref,
              batch_ref, w1, b1, w2, b2, w3, b3, out_ref, acc, cnt):
    i = pl.program_id(0)

    @pl.when(i == 0)
    def _():
        acc[...] = jnp.zeros_like(acc)
        cnt[...] = jnp.zeros_like(cnt)

    h, _ = _norm_act(deg_ref, sa, sb, ua, ub, b_ref, lns_ref, lnb_ref)
    h = h + hp_ref[...]
    seg = batch_ref[...]
    mask = (lax.broadcasted_iota(jnp.int32, (TN, NB), 1) == seg
            ).astype(jnp.float32)
    acc[...] += lax.dot_general(mask, h, (((0,), (0,)), ((), ())),
                                preferred_element_type=jnp.float32)
    cnt[...] += lax.dot_general(mask, jnp.ones((TN, 1), jnp.float32),
                                (((0,), (0,)), ((), ())),
                                preferred_element_type=jnp.float32)

    @pl.when(i == GI - 1)
    def _():
        pooled = acc[...] / jnp.maximum(cnt[...], 1.0)
        g = jnp.maximum(jnp.dot(pooled, w1[...],
                                preferred_element_type=jnp.float32)
                        + b1[...], 0.0)
        g = jnp.maximum(jnp.dot(g, w2[...],
                                preferred_element_type=jnp.float32)
                        + b2[...], 0.0)
        out_ref[...] = jnp.dot(g, w3[...],
                               preferred_element_type=jnp.float32) + b3[...]


def _node_spec(shape=(TN, H)):
    return pl.BlockSpec(shape, lambda i, c: (i, 0))


def _half_spec():
    return pl.BlockSpec((TN, HF), lambda i, c: (c * GI + i, 0))


def _halves_specs():
    a = pl.BlockSpec((TN, HF), lambda i: (i, 0))
    b = pl.BlockSpec((TN, HF), lambda i: (GI + i, 0))
    return a, b


_pre0 = pl.pallas_call(
    _pre0_body,
    grid=(GI, NC),
    in_specs=[
        pl.BlockSpec((TN, 1), lambda i, c: (i, 0)),
        pl.BlockSpec((TN, H), lambda i, c: (i, 0)),
        pl.BlockSpec((H, HF), lambda i, c: (0, c)),
    ],
    out_specs=_half_spec(),
    out_shape=jax.ShapeDtypeStruct((NC * NP, HF), jnp.float32),
    name="tc_pre0",
)


def _make_mid(residual):
    body = _mid_body_res if residual else _mid_body_nores
    sa2 = pl.BlockSpec((TN, HF), lambda i, c: (i, 0))
    sb2 = pl.BlockSpec((TN, HF), lambda i, c: (GI + i, 0))
    row = pl.BlockSpec((1, H), lambda i, c: (0, 0))
    specs = [
        pl.BlockSpec((TN, 1), lambda i, c: (i, 0)),
        sa2, sb2,
        pl.BlockSpec((TN, HF), lambda i, c: (i, 0)),
        pl.BlockSpec((TN, HF), lambda i, c: (GI + i, 0)),
        row, row, row,
    ]
    if residual:
        specs.append(pl.BlockSpec((TN, H), lambda i, c: (i, 0)))
    specs.append(pl.BlockSpec((H, HF), lambda i, c: (0, c)))
    return pl.pallas_call(
        body,
        grid=(GI, NC),
        in_specs=specs,
        out_specs=[
            pl.BlockSpec((TN, H), lambda i, c: (i, 0)),
            _half_spec(),
        ],
        out_shape=[
            jax.ShapeDtypeStruct((NP, H), jnp.float32),
            jax.ShapeDtypeStruct((NC * NP, HF), jnp.float32),
        ],
        name="tc_mid_res" if residual else "tc_mid",
    )


_mid0 = _make_mid(False)
_mid1 = _make_mid(True)

_sa1, _sb1 = _halves_specs()
_row1 = pl.BlockSpec((1, H), lambda i: (0, 0))
_fin = pl.pallas_call(
    _fin_body,
    grid=(GI,),
    in_specs=[
        pl.BlockSpec((TN, 1), lambda i: (i, 0)),
        _sa1, _sb1,
        pl.BlockSpec((TN, HF), lambda i: (i, 0)),
        pl.BlockSpec((TN, HF), lambda i: (GI + i, 0)),
        _row1, _row1, _row1,
        pl.BlockSpec((TN, H), lambda i: (i, 0)),
        pl.BlockSpec((TN, 1), lambda i: (i, 0)),
        pl.BlockSpec((H, H), lambda i: (0, 0)),
        pl.BlockSpec((1, H), lambda i: (0, 0)),
        pl.BlockSpec((H, HF), lambda i: (0, 0)),
        pl.BlockSpec((1, HF), lambda i: (0, 0)),
        pl.BlockSpec((HF, OUT), lambda i: (0, 0)),
        pl.BlockSpec((1, OUT), lambda i: (0, 0)),
    ],
    out_specs=pl.BlockSpec((NB, OUT), lambda i: (0, 0)),
    out_shape=jax.ShapeDtypeStruct((NB, OUT), jnp.float32),
    scratch_shapes=[
        pltpu.VMEM((NB, H), jnp.float32),
        pltpu.VMEM((NB, 1), jnp.float32),
    ],
    name="tc_final",
)


def kernel(x, edge_index, batch,
           conv_W0, conv_b0, conv_W1, conv_b1, conv_W2, conv_b2,
           ln_s0, ln_b0, ln_s1, ln_b1, ln_s2, ln_b2,
           gh_W1, gh_b1, gh_W2, gh_b2, gh_W3, gh_b3):
    src = edge_index[0]
    dst = edge_index[1]
    xp = jnp.pad(x, ((0, NP - N), (0, 0)))
    batch_p = jnp.pad(batch, (0, NP - N), constant_values=NB).reshape(NP, 1)

    deg = _deg_sc(dst).reshape(NP, 1)
    u0 = _pre0(deg, xp, conv_W0)
    s0 = _edge_sc(u0, src, dst)
    h0, u1 = _mid0(deg, s0, s0, u0, u0,
                   conv_b0.reshape(1, H), ln_s0.reshape(1, H),
                   ln_b0.reshape(1, H), conv_W1)
    s1 = _edge_sc(u1, src, dst)
    h1, u2 = _mid1(deg, s1, s1, u1, u1,
                   conv_b1.reshape(1, H), ln_s1.reshape(1, H),
                   ln_b1.reshape(1, H), h0, conv_W2)
    s2 = _edge_sc(u2, src, dst)
    out = _fin(deg, s2, s2, u2, u2,
               conv_b2.reshape(1, H), ln_s2.reshape(1, H),
               ln_b2.reshape(1, H), h1, batch_p,
               gh_W1, gh_b1.reshape(1, H), gh_W2, gh_b2.reshape(1, HF),
               gh_W3, gh_b3.reshape(1, OUT))
    return out


# speedup vs baseline: 16.5189x; 1.1174x over previous
import functools

import jax
import jax.numpy as jnp
from jax import lax
from jax.experimental import pallas as pl
from jax.experimental.pallas import tpu as pltpu
from jax.experimental.pallas import tpu_sc as plsc

N = 10000
NP = 10240
E = 160000
H = 256
HF = 128
NB = 64
OUT = 12
NC, NS = 2, 16
TN = 1024
GI = NP // TN
EPT = E // NS
KCH = 80
NCHUNK = EPT // KCH
RPT = NP // NS
DW = 16
RING = 3


def _deg_body(dst_hbm, out_hbm, dA, dB, ones_v, acc, semDA, semDB):
    c = lax.axis_index("c")
    s = lax.axis_index("s")

    @pl.when(c == 0)
    def _():
        zv = jnp.zeros((16,), jnp.float32)
        for j in range(KCH // 16):
            ones_v[pl.ds(16 * j, 16)] = zv
        for t in range(RPT // KCH):
            pltpu.sync_copy(ones_v, acc.at[pl.ds(s * RPT + t * KCH, KCH)])
        ov = jnp.ones((16,), jnp.float32)
        for j in range(KCH // 16):
            ones_v[pl.ds(16 * j, 16)] = ov
        plsc.subcore_barrier()

        base = s * EPT
        bufs = ((dA, semDA), (dB, semDB))

        def _issue(k, parity):
            d, semD = bufs[parity]
            pltpu.async_copy(dst_hbm.at[pl.ds(base + k * KCH, KCH)], d, semD)

        def _wait_scatter(parity):
            d, semD = bufs[parity]
            pltpu.make_async_copy(dst_hbm.at[pl.ds(0, KCH)], d, semD).wait()
            pltpu.sync_copy(ones_v, acc.at[d], add=True)

        _issue(0, 0)

        def _pair(i, carry):
            for p in range(2):
                k = 2 * i + p
                _issue(k + 1, 1 - p)
                _wait_scatter(p)
            return carry

        lax.fori_loop(0, NCHUNK // 2, _pair, 0)
        _wait_scatter(0)
        plsc.subcore_barrier()
        pltpu.sync_copy(acc.at[pl.ds(s * RPT, RPT)],
                        out_hbm.at[pl.ds(s * RPT, RPT)])


def _edge_body(u_hbm, src_hbm, dst_hbm, out_hbm, sidx, *rest):
    dbufs = rest[0:RING]
    rbufs = rest[RING:2 * RING]
    acc = rest[2 * RING]
    semG = rest[2 * RING + 1:3 * RING + 1]
    semD = rest[3 * RING + 1:4 * RING + 1]
    semS = rest[4 * RING + 1:5 * RING + 1]
    rowsA = rbufs[0]
    c = lax.axis_index("c")
    s = lax.axis_index("s")
    zv = jnp.zeros((16,), jnp.float32)

    def _zrow(r, carry):
        for j in range(HF // 16):
            rowsA[r, pl.ds(16 * j, 16)] = zv
        return carry

    lax.fori_loop(0, KCH, _zrow, 0)
    for t in range(RPT // KCH):
        pltpu.sync_copy(rowsA, acc.at[pl.ds(s * RPT + t * KCH, KCH)])

    base = s * EPT
    pltpu.sync_copy(src_hbm.at[pl.ds(base, EPT)], sidx)
    off = c * NP

    def _off(g, carry):
        sl = pl.ds(16 * g, 16)
        sidx[sl] = sidx[sl] + off
        return carry

    lax.fori_loop(0, EPT // 16, _off, 0)
    plsc.subcore_barrier()


    def _issue(k, parity):
        pltpu.async_copy(dst_hbm.at[pl.ds(base + k * KCH, KCH)],
                         dbufs[parity], semD[parity])
        pltpu.async_copy(u_hbm.at[sidx.at[pl.ds(k * KCH, KCH)]],
                         rbufs[parity], semG[parity])

    def _wait_in(parity):
        pltpu.make_async_copy(dst_hbm.at[pl.ds(0, KCH)],
                              dbufs[parity], semD[parity]).wait()
        pltpu.make_async_copy(u_hbm.at[pl.ds(0, KCH)],
                              rbufs[parity], semG[parity]).wait()

    def _scatter(parity):
        pltpu.async_copy(rbufs[parity], acc.at[dbufs[parity]],
                         semS[parity], add=True)

    def _wait_scat(parity):
        pltpu.make_async_copy(u_hbm.at[pl.ds(0, KCH)],
                              rbufs[parity], semS[parity]).wait()

    for k in range(RING - 1):
        _issue(k, k)

    def _group(g, carry):
        for p in range(RING):
            k = RING * g + p
            _wait_in(p)
            _scatter(p)
            kn = k + RING - 1
            pn = (p + RING - 1) % RING

            @pl.when(kn < NCHUNK)
            def _():
                @pl.when(k >= 1)
                def _():
                    _wait_scat(pn)

                _issue(kn, pn)
        return carry

    lax.fori_loop(0, NCHUNK // RING, _group, 0)
    for k in range((NCHUNK // RING) * RING, NCHUNK):
        _wait_in(k % RING)
        _scatter(k % RING)
    for p in range(RING):
        _wait_scat(p)

    plsc.subcore_barrier()
    pltpu.sync_copy(acc.at[pl.ds(s * RPT, RPT)],
                    out_hbm.at[pl.ds(c * NP + s * RPT, RPT)])


@functools.cache
def _sc_kernels():
    mesh = plsc.VectorSubcoreMesh(
        core_axis_name="c", subcore_axis_name="s",
        num_cores=NC, num_subcores=NS)
    deg = pl.kernel(
        _deg_body,
        out_type=jax.ShapeDtypeStruct((NP,), jnp.float32),
        mesh=mesh,
        scratch_types=[
            pltpu.VMEM((KCH,), jnp.int32),
            pltpu.VMEM((KCH,), jnp.int32),
            pltpu.VMEM((KCH,), jnp.float32),
            pltpu.VMEM_SHARED((NP,), jnp.float32),
            pltpu.SemaphoreType.DMA,
            pltpu.SemaphoreType.DMA,
        ],
        name="sc_degree",
    )
    edge = pl.kernel(
        _edge_body,
        out_type=jax.ShapeDtypeStruct((NC * NP, HF), jnp.float32),
        mesh=mesh,
        scratch_types=(
            [pltpu.VMEM((EPT,), jnp.int32)]
            + [pltpu.VMEM((KCH,), jnp.int32) for _ in range(RING)]
            + [pltpu.VMEM((KCH, HF), jnp.float32) for _ in range(RING)]
            + [pltpu.VMEM_SHARED((NP, HF), jnp.float32)]
            + [pltpu.SemaphoreType.DMA for _ in range(3 * RING)]
        ),
        name="sc_edge_pass",
    )
    return deg, edge


def _deg_sc(dst):
    return _sc_kernels()[0](dst)


def _edge_sc(u, src, dst):
    return _sc_kernels()[1](u, src, dst)


def _dinv_of(deg_ref):
    return lax.rsqrt(deg_ref[:, :1] + 1.0)


def _pre0_body(deg_ref, x_ref, w_ref, u_ref):
    dinv = _dinv_of(deg_ref)
    u_ref[...] = jnp.dot(x_ref[...], w_ref[...],
                         preferred_element_type=jnp.float32) * dinv


def _norm_act(deg_ref, sa, sb, ua, ub, b_ref, lns_ref, lnb_ref):
    dinv = _dinv_of(deg_ref)
    t = jnp.concatenate([sa[...] + ua[...], sb[...] + ub[...]], axis=1)
    t = t * dinv + b_ref[...]
    mu = jnp.mean(t, axis=1, keepdims=True)
    var = jnp.mean((t - mu) ** 2, axis=1, keepdims=True)
    y = (t - mu) * lax.rsqrt(var + 1e-5) * lns_ref[...] + lnb_ref[...]
    return jnp.maximum(y, 0.0), dinv


def _mid_body_nores(deg_ref, sa, sb, ua, ub, b_ref, lns_ref, lnb_ref, w_ref,
                    h_out, u_out):
    h, dinv = _norm_act(deg_ref, sa, sb, ua, ub, b_ref, lns_ref, lnb_ref)
    h_out[...] = h
    u_out[...] = jnp.dot(h, w_ref[...],
                         preferred_element_type=jnp.float32) * dinv


def _mid_body_res(deg_ref, sa, sb, ua, ub, b_ref, lns_ref, lnb_ref, hp_ref,
                  w_ref, h_out, u_out):
    h, dinv = _norm_act(deg_ref, sa, sb, ua, ub, b_ref, lns_ref, lnb_ref)
    h = h + hp_ref[...]
    h_out[...] = h
    u_out[...] = jnp.dot(h, w_ref[...],
                         preferred_element_type=jnp.float32) * dinv


def _fin_body(deg_ref, sa, sb, ua, ub, b_ref, lns_ref, lnb_ref, hp_ref,
              batch_ref, w1, b1, w2, b2, w3, b3, out_ref, acc, cnt):
    i = pl.program_id(0)

    @pl.when(i == 0)
    def _():
        acc[...] = jnp.zeros_like(acc)
        cnt[...] = jnp.zeros_like(cnt)

    h, _ = _norm_act(deg_ref, sa, sb, ua, ub, b_ref, lns_ref, lnb_ref)
    h = h + hp_ref[...]
    seg = batch_ref[...]
    mask = (lax.broadcasted_iota(jnp.int32, (TN, NB), 1) == seg
            ).astype(jnp.float32)
    acc[...] += lax.dot_general(mask, h, (((0,), (0,)), ((), ())),
                                preferred_element_type=jnp.float32)
    cnt[...] += lax.dot_general(mask, jnp.ones((TN, 1), jnp.float32),
                                (((0,), (0,)), ((), ())),
                                preferred_element_type=jnp.float32)

    @pl.when(i == GI - 1)
    def _():
        pooled = acc[...] / jnp.maximum(cnt[...], 1.0)
        g = jnp.maximum(jnp.dot(pooled, w1[...],
                                preferred_element_type=jnp.float32)
                        + b1[...], 0.0)
        g = jnp.maximum(jnp.dot(g, w2[...],
                                preferred_element_type=jnp.float32)
                        + b2[...], 0.0)
        out_ref[...] = jnp.dot(g, w3[...],
                               preferred_element_type=jnp.float32) + b3[...]


def _node_spec(shape=(TN, H)):
    return pl.BlockSpec(shape, lambda i, c: (i, 0))


def _half_spec():
    return pl.BlockSpec((TN, HF), lambda i, c: (c * GI + i, 0))


def _halves_specs():
    a = pl.BlockSpec((TN, HF), lambda i: (i, 0))
    b = pl.BlockSpec((TN, HF), lambda i: (GI + i, 0))
    return a, b


_pre0 = pl.pallas_call(
    _pre0_body,
    grid=(GI, NC),
    in_specs=[
        pl.BlockSpec((TN, 1), lambda i, c: (i, 0)),
        pl.BlockSpec((TN, H), lambda i, c: (i, 0)),
        pl.BlockSpec((H, HF), lambda i, c: (0, c)),
    ],
    out_specs=_half_spec(),
    out_shape=jax.ShapeDtypeStruct((NC * NP, HF), jnp.float32),
    name="tc_pre0",
)


def _make_mid(residual):
    body = _mid_body_res if residual else _mid_body_nores
    sa2 = pl.BlockSpec((TN, HF), lambda i, c: (i, 0))
    sb2 = pl.BlockSpec((TN, HF), lambda i, c: (GI + i, 0))
    row = pl.BlockSpec((1, H), lambda i, c: (0, 0))
    specs = [
        pl.BlockSpec((TN, 1), lambda i, c: (i, 0)),
        sa2, sb2,
        pl.BlockSpec((TN, HF), lambda i, c: (i, 0)),
        pl.BlockSpec((TN, HF), lambda i, c: (GI + i, 0)),
        row, row, row,
    ]
    if residual:
        specs.append(pl.BlockSpec((TN, H), lambda i, c: (i, 0)))
    specs.append(pl.BlockSpec((H, HF), lambda i, c: (0, c)))
    return pl.pallas_call(
        body,
        grid=(GI, NC),
        in_specs=specs,
        out_specs=[
            pl.BlockSpec((TN, H), lambda i, c: (i, 0)),
            _half_spec(),
        ],
        out_shape=[
            jax.ShapeDtypeStruct((NP, H), jnp.float32),
            jax.ShapeDtypeStruct((NC * NP, HF), jnp.float32),
        ],
        name="tc_mid_res" if residual else "tc_mid",
    )


_mid0 = _make_mid(False)
_mid1 = _make_mid(True)

_sa1, _sb1 = _halves_specs()
_row1 = pl.BlockSpec((1, H), lambda i: (0, 0))
_fin = pl.pallas_call(
    _fin_body,
    grid=(GI,),
    in_specs=[
        pl.BlockSpec((TN, 1), lambda i: (i, 0)),
        _sa1, _sb1,
        pl.BlockSpec((TN, HF), lambda i: (i, 0)),
        pl.BlockSpec((TN, HF), lambda i: (GI + i, 0)),
        _row1, _row1, _row1,
        pl.BlockSpec((TN, H), lambda i: (i, 0)),
        pl.BlockSpec((TN, 1), lambda i: (i, 0)),
        pl.BlockSpec((H, H), lambda i: (0, 0)),
        pl.BlockSpec((1, H), lambda i: (0, 0)),
        pl.BlockSpec((H, HF), lambda i: (0, 0)),
        pl.BlockSpec((1, HF), lambda i: (0, 0)),
        pl.BlockSpec((HF, OUT), lambda i: (0, 0)),
        pl.BlockSpec((1, OUT), lambda i: (0, 0)),
    ],
    out_specs=pl.BlockSpec((NB, OUT), lambda i: (0, 0)),
    out_shape=jax.ShapeDtypeStruct((NB, OUT), jnp.float32),
    scratch_shapes=[
        pltpu.VMEM((NB, H), jnp.float32),
        pltpu.VMEM((NB, 1), jnp.float32),
    ],
    name="tc_final",
)


def kernel(x, edge_index, batch,
           conv_W0, conv_b0, conv_W1, conv_b1, conv_W2, conv_b2,
           ln_s0, ln_b0, ln_s1, ln_b1, ln_s2, ln_b2,
           gh_W1, gh_b1, gh_W2, gh_b2, gh_W3, gh_b3):
    src = edge_index[0]
    dst = edge_index[1]
    xp = jnp.pad(x, ((0, NP - N), (0, 0)))
    batch_p = jnp.pad(batch, (0, NP - N), constant_values=NB).reshape(NP, 1)

    deg = _deg_sc(dst).reshape(NP, 1)
    u0 = _pre0(deg, xp, conv_W0)
    s0 = _edge_sc(u0, src, dst)
    h0, u1 = _mid0(deg, s0, s0, u0, u0,
                   conv_b0.reshape(1, H), ln_s0.reshape(1, H),
                   ln_b0.reshape(1, H), conv_W1)
    s1 = _edge_sc(u1, src, dst)
    h1, u2 = _mid1(deg, s1, s1, u1, u1,
                   conv_b1.reshape(1, H), ln_s1.reshape(1, H),
                   ln_b1.reshape(1, H), h0, conv_W2)
    s2 = _edge_sc(u2, src, dst)
    out = _fin(deg, s2, s2, u2, u2,
               conv_b2.reshape(1, H), ln_s2.reshape(1, H),
               ln_b2.reshape(1, H), h1, batch_p,
               gh_W1, gh_b1.reshape(1, H), gh_W2, gh_b2.reshape(1, HF),
               gh_W3, gh_b3.reshape(1, OUT))
    return out


# speedup vs baseline: 16.5975x; 1.0048x over previous
import functools

import jax
import jax.numpy as jnp
from jax import lax
from jax.experimental import pallas as pl
from jax.experimental.pallas import tpu as pltpu
from jax.experimental.pallas import tpu_sc as plsc

N = 10000
NP = 10240
E = 160000
H = 256
HF = 128
NB = 64
OUT = 12
NC, NS = 2, 16
TN = 1024
GI = NP // TN
EPT = E // NS
KCH = 80
NCHUNK = EPT // KCH
RPT = NP // NS
DW = 16
RING = 3


def _deg_body(dst_hbm, out_hbm, dA, dB, ones_v, acc, semDA, semDB):
    c = lax.axis_index("c")
    s = lax.axis_index("s")

    @pl.when(c == 0)
    def _():
        zv = jnp.zeros((16,), jnp.float32)
        for j in range(KCH // 16):
            ones_v[pl.ds(16 * j, 16)] = zv
        for t in range(RPT // KCH):
            pltpu.sync_copy(ones_v, acc.at[pl.ds(s * RPT + t * KCH, KCH)])
        ov = jnp.ones((16,), jnp.float32)
        for j in range(KCH // 16):
            ones_v[pl.ds(16 * j, 16)] = ov
        plsc.subcore_barrier()

        base = s * EPT
        bufs = ((dA, semDA), (dB, semDB))

        def _issue(k, parity):
            d, semD = bufs[parity]
            pltpu.async_copy(dst_hbm.at[pl.ds(base + k * KCH, KCH)], d, semD)

        def _wait_scatter(parity):
            d, semD = bufs[parity]
            pltpu.make_async_copy(dst_hbm.at[pl.ds(0, KCH)], d, semD).wait()
            pltpu.sync_copy(ones_v, acc.at[d], add=True)

        _issue(0, 0)

        def _pair(i, carry):
            for p in range(2):
                k = 2 * i + p
                _issue(k + 1, 1 - p)
                _wait_scatter(p)
            return carry

        lax.fori_loop(0, NCHUNK // 2, _pair, 0)
        _wait_scatter(0)
        plsc.subcore_barrier()
        pltpu.sync_copy(acc.at[pl.ds(s * RPT, RPT)],
                        out_hbm.at[pl.ds(s * RPT, RPT)])


def _edge_body(u_hbm, src_hbm, dst_hbm, out_hbm, sidx, *rest):
    dbufs = rest[0:RING]
    rbufs = rest[RING:2 * RING]
    acc = rest[2 * RING]
    semG = rest[2 * RING + 1:3 * RING + 1]
    semD = rest[3 * RING + 1:4 * RING + 1]
    semS = rest[4 * RING + 1:5 * RING + 1]
    rowsA = rbufs[0]
    c = lax.axis_index("c")
    s = lax.axis_index("s")
    zv = jnp.zeros((16,), jnp.float32)

    def _zrow(r, carry):
        for j in range(HF // 16):
            rowsA[r, pl.ds(16 * j, 16)] = zv
        return carry

    lax.fori_loop(0, KCH, _zrow, 0)
    for t in range(RPT // KCH):
        pltpu.sync_copy(rowsA, acc.at[pl.ds(s * RPT + t * KCH, KCH)])

    base = s * EPT
    pltpu.sync_copy(src_hbm.at[pl.ds(base, EPT)], sidx)
    off = c * NP

    def _off(g, carry):
        sl = pl.ds(16 * g, 16)
        sidx[sl] = sidx[sl] + off
        return carry

    lax.fori_loop(0, EPT // 16, _off, 0)
    plsc.subcore_barrier()


    def _issue(k, parity):
        pltpu.async_copy(dst_hbm.at[pl.ds(base + k * KCH, KCH)],
                         dbufs[parity], semD[parity])
        pltpu.async_copy(u_hbm.at[sidx.at[pl.ds(k * KCH, KCH)]],
                         rbufs[parity], semG[parity])

    def _wait_in(parity):
        pltpu.make_async_copy(dst_hbm.at[pl.ds(0, KCH)],
                              dbufs[parity], semD[parity]).wait()
        pltpu.make_async_copy(u_hbm.at[pl.ds(0, KCH)],
                              rbufs[parity], semG[parity]).wait()

    def _scatter(parity):
        pltpu.async_copy(rbufs[parity], acc.at[dbufs[parity]],
                         semS[parity], add=True)

    def _wait_scat(parity):
        pltpu.make_async_copy(u_hbm.at[pl.ds(0, KCH)],
                              rbufs[parity], semS[parity]).wait()

    for k in range(RING - 1):
        _issue(k, k)

    def _group(g, carry):
        for p in range(RING):
            k = RING * g + p
            _wait_in(p)
            _scatter(p)
            kn = k + RING - 1
            pn = (p + RING - 1) % RING

            @pl.when(kn < NCHUNK)
            def _():
                @pl.when(k >= 1)
                def _():
                    _wait_scat(pn)

                _issue(kn, pn)
        return carry

    lax.fori_loop(0, NCHUNK // RING, _group, 0)
    for k in range((NCHUNK // RING) * RING, NCHUNK):
        _wait_in(k % RING)
        _scatter(k % RING)
    for p in range(RING):
        _wait_scat(p)

    plsc.subcore_barrier()
    pltpu.sync_copy(acc.at[pl.ds(s * RPT, RPT)],
                    out_hbm.at[pl.ds(c * NP + s * RPT, RPT)])


@functools.cache
def _sc_kernels():
    mesh = plsc.VectorSubcoreMesh(
        core_axis_name="c", subcore_axis_name="s",
        num_cores=NC, num_subcores=NS)
    deg = pl.kernel(
        _deg_body,
        out_type=jax.ShapeDtypeStruct((NP,), jnp.float32),
        mesh=mesh,
        scratch_types=[
            pltpu.VMEM((KCH,), jnp.int32),
            pltpu.VMEM((KCH,), jnp.int32),
            pltpu.VMEM((KCH,), jnp.float32),
            pltpu.VMEM_SHARED((NP,), jnp.float32),
            pltpu.SemaphoreType.DMA,
            pltpu.SemaphoreType.DMA,
        ],
        name="sc_degree",
    )
    edge = pl.kernel(
        _edge_body,
        out_type=jax.ShapeDtypeStruct((NC * NP, HF), jnp.float32),
        mesh=mesh,
        scratch_types=(
            [pltpu.VMEM((EPT,), jnp.int32)]
            + [pltpu.VMEM((KCH,), jnp.int32) for _ in range(RING)]
            + [pltpu.VMEM((KCH, HF), jnp.float32) for _ in range(RING)]
            + [pltpu.VMEM_SHARED((NP, HF), jnp.float32)]
            + [pltpu.SemaphoreType.DMA for _ in range(3 * RING)]
        ),
        name="sc_edge_pass",
    )
    return deg, edge


def _deg_sc(dst):
    return _sc_kernels()[0](dst)


def _edge_sc(u, src, dst):
    return _sc_kernels()[1](u, src, dst)


def _dinv_of(deg_ref):
    return lax.rsqrt(deg_ref[:, :1] + 1.0)


def _bdot(a, b):
    return jnp.dot(a.astype(jnp.bfloat16), b.astype(jnp.bfloat16),
                   preferred_element_type=jnp.float32)


def _pre0_body(deg_ref, x_ref, w_ref, u_ref):
    dinv = _dinv_of(deg_ref)
    u_ref[...] = _bdot(x_ref[...], w_ref[...]) * dinv


def _norm_act(deg_ref, sa, sb, ua, ub, b_ref, lns_ref, lnb_ref):
    dinv = _dinv_of(deg_ref)
    t = jnp.concatenate([sa[...] + ua[...], sb[...] + ub[...]], axis=1)
    t = t * dinv + b_ref[...]
    mu = jnp.mean(t, axis=1, keepdims=True)
    var = jnp.mean((t - mu) ** 2, axis=1, keepdims=True)
    y = (t - mu) * lax.rsqrt(var + 1e-5) * lns_ref[...] + lnb_ref[...]
    return jnp.maximum(y, 0.0), dinv


def _mid_body_nores(deg_ref, sa, sb, ua, ub, b_ref, lns_ref, lnb_ref, w_ref,
                    h_out, u_out):
    h, dinv = _norm_act(deg_ref, sa, sb, ua, ub, b_ref, lns_ref, lnb_ref)
    h_out[...] = h
    u_out[...] = _bdot(h, w_ref[...]) * dinv


def _mid_body_res(deg_ref, sa, sb, ua, ub, b_ref, lns_ref, lnb_ref, hp_ref,
                  w_ref, h_out, u_out):
    h, dinv = _norm_act(deg_ref, sa, sb, ua, ub, b_ref, lns_ref, lnb_ref)
    h = h + hp_ref[...]
    h_out[...] = h
    u_out[...] = _bdot(h, w_ref[...]) * dinv


def _fin_body(deg_ref, sa, sb, ua, ub, b_ref, lns_ref, lnb_ref, hp_ref,
              batch_ref, w1, b1, w2, b2, w3, b3, out_ref, acc, cnt):
    i = pl.program_id(0)

    @pl.when(i == 0)
    def _():
        acc[...] = jnp.zeros_like(acc)
        cnt[...] = jnp.zeros_like(cnt)

    h, _ = _norm_act(deg_ref, sa, sb, ua, ub, b_ref, lns_ref, lnb_ref)
    h = h + hp_ref[...]
    rid = lax.broadcasted_iota(jnp.int32, (TN, 1), 0) + i * TN
    h = jnp.where(rid < N, h, 0.0)
    seg = batch_ref[...]
    mask = ((lax.broadcasted_iota(jnp.int32, (TN, NB), 1) == seg)
            & (rid < N)).astype(jnp.float32)
    acc[...] += lax.dot_general(mask, h, (((0,), (0,)), ((), ())),
                                preferred_element_type=jnp.float32)
    cnt[...] += lax.dot_general(mask, jnp.ones((TN, 1), jnp.float32),
                                (((0,), (0,)), ((), ())),
                                preferred_element_type=jnp.float32)

    @pl.when(i == GI - 1)
    def _():
        pooled = acc[...] / jnp.maximum(cnt[...], 1.0)
        g = jnp.maximum(jnp.dot(pooled, w1[...],
                                preferred_element_type=jnp.float32)
                        + b1[...], 0.0)
        g = jnp.maximum(jnp.dot(g, w2[...],
                                preferred_element_type=jnp.float32)
                        + b2[...], 0.0)
        out_ref[...] = jnp.dot(g, w3[...],
                               preferred_element_type=jnp.float32) + b3[...]


def _node_spec(shape=(TN, H)):
    return pl.BlockSpec(shape, lambda i, c: (i, 0))


def _half_spec():
    return pl.BlockSpec((TN, HF), lambda i, c: (c * GI + i, 0))


def _halves_specs():
    a = pl.BlockSpec((TN, HF), lambda i: (i, 0))
    b = pl.BlockSpec((TN, HF), lambda i: (GI + i, 0))
    return a, b


_pre0 = pl.pallas_call(
    _pre0_body,
    grid=(GI, NC),
    in_specs=[
        pl.BlockSpec((TN, 1), lambda i, c: (i, 0)),
        pl.BlockSpec((TN, H), lambda i, c: (i, 0)),
        pl.BlockSpec((H, HF), lambda i, c: (0, c)),
    ],
    out_specs=_half_spec(),
    out_shape=jax.ShapeDtypeStruct((NC * NP, HF), jnp.float32),
    name="tc_pre0",
)


def _make_mid(residual):
    body = _mid_body_res if residual else _mid_body_nores
    sa2 = pl.BlockSpec((TN, HF), lambda i, c: (i, 0))
    sb2 = pl.BlockSpec((TN, HF), lambda i, c: (GI + i, 0))
    row = pl.BlockSpec((1, H), lambda i, c: (0, 0))
    specs = [
        pl.BlockSpec((TN, 1), lambda i, c: (i, 0)),
        sa2, sb2,
        pl.BlockSpec((TN, HF), lambda i, c: (i, 0)),
        pl.BlockSpec((TN, HF), lambda i, c: (GI + i, 0)),
        row, row, row,
    ]
    if residual:
        specs.append(pl.BlockSpec((TN, H), lambda i, c: (i, 0)))
    specs.append(pl.BlockSpec((H, HF), lambda i, c: (0, c)))
    return pl.pallas_call(
        body,
        grid=(GI, NC),
        in_specs=specs,
        out_specs=[
            pl.BlockSpec((TN, H), lambda i, c: (i, 0)),
            _half_spec(),
        ],
        out_shape=[
            jax.ShapeDtypeStruct((NP, H), jnp.float32),
            jax.ShapeDtypeStruct((NC * NP, HF), jnp.float32),
        ],
        name="tc_mid_res" if residual else "tc_mid",
    )


_mid0 = _make_mid(False)
_mid1 = _make_mid(True)

_sa1, _sb1 = _halves_specs()
_row1 = pl.BlockSpec((1, H), lambda i: (0, 0))
_fin = pl.pallas_call(
    _fin_body,
    grid=(GI,),
    in_specs=[
        pl.BlockSpec((TN, 1), lambda i: (i, 0)),
        _sa1, _sb1,
        pl.BlockSpec((TN, HF), lambda i: (i, 0)),
        pl.BlockSpec((TN, HF), lambda i: (GI + i, 0)),
        _row1, _row1, _row1,
        pl.BlockSpec((TN, H), lambda i: (i, 0)),
        pl.BlockSpec((TN, 1), lambda i: (i, 0)),
        pl.BlockSpec((H, H), lambda i: (0, 0)),
        pl.BlockSpec((1, H), lambda i: (0, 0)),
        pl.BlockSpec((H, HF), lambda i: (0, 0)),
        pl.BlockSpec((1, HF), lambda i: (0, 0)),
        pl.BlockSpec((HF, OUT), lambda i: (0, 0)),
        pl.BlockSpec((1, OUT), lambda i: (0, 0)),
    ],
    out_specs=pl.BlockSpec((NB, OUT), lambda i: (0, 0)),
    out_shape=jax.ShapeDtypeStruct((NB, OUT), jnp.float32),
    scratch_shapes=[
        pltpu.VMEM((NB, H), jnp.float32),
        pltpu.VMEM((NB, 1), jnp.float32),
    ],
    name="tc_final",
)


def kernel(x, edge_index, batch,
           conv_W0, conv_b0, conv_W1, conv_b1, conv_W2, conv_b2,
           ln_s0, ln_b0, ln_s1, ln_b1, ln_s2, ln_b2,
           gh_W1, gh_b1, gh_W2, gh_b2, gh_W3, gh_b3):
    src = edge_index[0]
    dst = edge_index[1]
    batch_p = batch.reshape(N, 1)

    deg = _deg_sc(dst).reshape(NP, 1)
    u0 = _pre0(deg, x, conv_W0)
    s0 = _edge_sc(u0, src, dst)
    h0, u1 = _mid0(deg, s0, s0, u0, u0,
                   conv_b0.reshape(1, H), ln_s0.reshape(1, H),
                   ln_b0.reshape(1, H), conv_W1)
    s1 = _edge_sc(u1, src, dst)
    h1, u2 = _mid1(deg, s1, s1, u1, u1,
                   conv_b1.reshape(1, H), ln_s1.reshape(1, H),
                   ln_b1.reshape(1, H), h0, conv_W2)
    s2 = _edge_sc(u2, src, dst)
    out = _fin(deg, s2, s2, u2, u2,
               conv_b2.reshape(1, H), ln_s2.reshape(1, H),
               ln_b2.reshape(1, H), h1, batch_p,
               gh_W1, gh_b1.reshape(1, H), gh_W2, gh_b2.reshape(1, HF),
               gh_W3, gh_b3.reshape(1, OUT))
    return out


# speedup vs baseline: 16.5984x; 1.0001x over previous
import functools

import jax
import jax.numpy as jnp
from jax import lax
from jax.experimental import pallas as pl
from jax.experimental.pallas import tpu as pltpu
from jax.experimental.pallas import tpu_sc as plsc

N = 10000
NP = 10240
E = 160000
H = 256
HF = 128
NB = 64
OUT = 12
NC, NS = 2, 16
TN = 1024
GI = NP // TN
EPT = E // NS
KCH = 80
NCHUNK = EPT // KCH
RPT = NP // NS
DW = 16
RING = 3


def _deg_body(dst_hbm, out_hbm, dA, dB, ones_v, acc, semDA, semDB,
              semSA, semSB):
    c = lax.axis_index("c")
    s = lax.axis_index("s")

    @pl.when(c == 0)
    def _():
        zv = jnp.zeros((16,), jnp.float32)
        for j in range(KCH // 16):
            ones_v[pl.ds(16 * j, 16)] = zv
        for t in range(RPT // KCH):
            pltpu.sync_copy(ones_v, acc.at[pl.ds(s * RPT + t * KCH, KCH)])
        ov = jnp.ones((16,), jnp.float32)
        for j in range(KCH // 16):
            ones_v[pl.ds(16 * j, 16)] = ov
        plsc.subcore_barrier()

        base = s * EPT
        bufs = ((dA, semDA, semSA), (dB, semDB, semSB))

        def _issue(k, parity):
            d, semD, _ = bufs[parity]
            pltpu.async_copy(dst_hbm.at[pl.ds(base + k * KCH, KCH)], d, semD)

        def _wait_scat(parity):
            d, _, semS = bufs[parity]
            pltpu.make_async_copy(ones_hack_src.at[pl.ds(0, KCH)], ones_v,
                                  semS).wait()

        def _step(k, parity):
            d, semD, semS = bufs[parity]
            pltpu.make_async_copy(dst_hbm.at[pl.ds(0, KCH)], d, semD).wait()
            pltpu.async_copy(ones_v, acc.at[d], semS, add=True)

        ones_hack_src = dst_hbm
        _issue(0, 0)

        def _pair(i, carry):
            for p in range(2):
                k = 2 * i + p
                @pl.when(k >= 1)
                def _():
                    _wait_scat(1 - p)

                _issue(k + 1, 1 - p)
                _step(k, p)
            return carry

        lax.fori_loop(0, NCHUNK // 2, _pair, 0)
        _step(NCHUNK - 1, 0)
        _wait_scat(0)
        _wait_scat(1)
        plsc.subcore_barrier()
        pltpu.sync_copy(acc.at[pl.ds(s * RPT, RPT)],
                        out_hbm.at[pl.ds(s * RPT, RPT)])


def _edge_body(u_hbm, src_hbm, dst_hbm, out_hbm, sidx, *rest):
    dbufs = rest[0:RING]
    rbufs = rest[RING:2 * RING]
    acc = rest[2 * RING]
    semG = rest[2 * RING + 1:3 * RING + 1]
    semD = rest[3 * RING + 1:4 * RING + 1]
    semS = rest[4 * RING + 1:5 * RING + 1]
    rowsA = rbufs[0]
    c = lax.axis_index("c")
    s = lax.axis_index("s")
    zv = jnp.zeros((16,), jnp.float32)

    def _zrow(r, carry):
        for j in range(HF // 16):
            rowsA[r, pl.ds(16 * j, 16)] = zv
        return carry

    lax.fori_loop(0, KCH, _zrow, 0)
    for t in range(RPT // KCH):
        pltpu.sync_copy(rowsA, acc.at[pl.ds(s * RPT + t * KCH, KCH)])

    base = s * EPT
    pltpu.sync_copy(src_hbm.at[pl.ds(base, EPT)], sidx)
    off = c * NP

    def _off(g, carry):
        sl = pl.ds(16 * g, 16)
        sidx[sl] = sidx[sl] + off
        return carry

    lax.fori_loop(0, EPT // 16, _off, 0)
    plsc.subcore_barrier()


    def _issue(k, parity):
        pltpu.async_copy(dst_hbm.at[pl.ds(base + k * KCH, KCH)],
                         dbufs[parity], semD[parity])
        pltpu.async_copy(u_hbm.at[sidx.at[pl.ds(k * KCH, KCH)]],
                         rbufs[parity], semG[parity])

    def _wait_in(parity):
        pltpu.make_async_copy(dst_hbm.at[pl.ds(0, KCH)],
                              dbufs[parity], semD[parity]).wait()
        pltpu.make_async_copy(u_hbm.at[pl.ds(0, KCH)],
                              rbufs[parity], semG[parity]).wait()

    def _scatter(parity):
        pltpu.async_copy(rbufs[parity], acc.at[dbufs[parity]],
                         semS[parity], add=True)

    def _wait_scat(parity):
        pltpu.make_async_copy(u_hbm.at[pl.ds(0, KCH)],
                              rbufs[parity], semS[parity]).wait()

    for k in range(RING - 1):
        _issue(k, k)

    def _group(g, carry):
        for p in range(RING):
            k = RING * g + p
            _wait_in(p)
            _scatter(p)
            kn = k + RING - 1
            pn = (p + RING - 1) % RING

            @pl.when(kn < NCHUNK)
            def _():
                @pl.when(k >= 1)
                def _():
                    _wait_scat(pn)

                _issue(kn, pn)
        return carry

    lax.fori_loop(0, NCHUNK // RING, _group, 0)
    for k in range((NCHUNK // RING) * RING, NCHUNK):
        _wait_in(k % RING)
        _scatter(k % RING)
    for p in range(RING):
        _wait_scat(p)

    plsc.subcore_barrier()
    pltpu.sync_copy(acc.at[pl.ds(s * RPT, RPT)],
                    out_hbm.at[pl.ds(c * NP + s * RPT, RPT)])


@functools.cache
def _sc_kernels():
    mesh = plsc.VectorSubcoreMesh(
        core_axis_name="c", subcore_axis_name="s",
        num_cores=NC, num_subcores=NS)
    deg = pl.kernel(
        _deg_body,
        out_type=jax.ShapeDtypeStruct((NP,), jnp.float32),
        mesh=mesh,
        scratch_types=[
            pltpu.VMEM((KCH,), jnp.int32),
            pltpu.VMEM((KCH,), jnp.int32),
            pltpu.VMEM((KCH,), jnp.float32),
            pltpu.VMEM_SHARED((NP,), jnp.float32),
            pltpu.SemaphoreType.DMA,
            pltpu.SemaphoreType.DMA,
            pltpu.SemaphoreType.DMA,
            pltpu.SemaphoreType.DMA,
        ],
        name="sc_degree",
    )
    edge = pl.kernel(
        _edge_body,
        out_type=jax.ShapeDtypeStruct((NC * NP, HF), jnp.float32),
        mesh=mesh,
        scratch_types=(
            [pltpu.VMEM((EPT,), jnp.int32)]
            + [pltpu.VMEM((KCH,), jnp.int32) for _ in range(RING)]
            + [pltpu.VMEM((KCH, HF), jnp.float32) for _ in range(RING)]
            + [pltpu.VMEM_SHARED((NP, HF), jnp.float32)]
            + [pltpu.SemaphoreType.DMA for _ in range(3 * RING)]
        ),
        name="sc_edge_pass",
    )
    return deg, edge


def _deg_sc(dst):
    return _sc_kernels()[0](dst)


def _edge_sc(u, src, dst):
    return _sc_kernels()[1](u, src, dst)


def _dinv_of(deg_ref):
    return lax.rsqrt(deg_ref[:, :1] + 1.0)


def _bdot(a, b):
    return jnp.dot(a.astype(jnp.bfloat16), b.astype(jnp.bfloat16),
                   preferred_element_type=jnp.float32)


def _pre0_body(deg_ref, x_ref, w_ref, u_ref):
    dinv = _dinv_of(deg_ref)
    u_ref[...] = _bdot(x_ref[...], w_ref[...]) * dinv


def _norm_act(deg_ref, sa, sb, ua, ub, b_ref, lns_ref, lnb_ref):
    dinv = _dinv_of(deg_ref)
    t = jnp.concatenate([sa[...] + ua[...], sb[...] + ub[...]], axis=1)
    t = t * dinv + b_ref[...]
    mu = jnp.mean(t, axis=1, keepdims=True)
    var = jnp.mean((t - mu) ** 2, axis=1, keepdims=True)
    y = (t - mu) * lax.rsqrt(var + 1e-5) * lns_ref[...] + lnb_ref[...]
    return jnp.maximum(y, 0.0), dinv


def _mid_body_nores(deg_ref, sa, sb, ua, ub, b_ref, lns_ref, lnb_ref, w_ref,
                    h_out, u_out):
    h, dinv = _norm_act(deg_ref, sa, sb, ua, ub, b_ref, lns_ref, lnb_ref)
    h_out[...] = h
    u_out[...] = _bdot(h, w_ref[...]) * dinv


def _mid_body_res(deg_ref, sa, sb, ua, ub, b_ref, lns_ref, lnb_ref, hp_ref,
                  w_ref, h_out, u_out):
    h, dinv = _norm_act(deg_ref, sa, sb, ua, ub, b_ref, lns_ref, lnb_ref)
    h = h + hp_ref[...]
    h_out[...] = h
    u_out[...] = _bdot(h, w_ref[...]) * dinv


def _fin_body(deg_ref, sa, sb, ua, ub, b_ref, lns_ref, lnb_ref, hp_ref,
              batch_ref, w1, b1, w2, b2, w3, b3, out_ref, acc, cnt):
    i = pl.program_id(0)

    @pl.when(i == 0)
    def _():
        acc[...] = jnp.zeros_like(acc)
        cnt[...] = jnp.zeros_like(cnt)

    h, _ = _norm_act(deg_ref, sa, sb, ua, ub, b_ref, lns_ref, lnb_ref)
    h = h + hp_ref[...]
    rid = lax.broadcasted_iota(jnp.int32, (TN, 1), 0) + i * TN
    h = jnp.where(rid < N, h, 0.0)
    seg = batch_ref[...]
    mask = ((lax.broadcasted_iota(jnp.int32, (TN, NB), 1) == seg)
            & (rid < N)).astype(jnp.float32)
    acc[...] += lax.dot_general(mask, h, (((0,), (0,)), ((), ())),
                                preferred_element_type=jnp.float32)
    cnt[...] += lax.dot_general(mask, jnp.ones((TN, 1), jnp.float32),
                                (((0,), (0,)), ((), ())),
                                preferred_element_type=jnp.float32)

    @pl.when(i == GI - 1)
    def _():
        pooled = acc[...] / jnp.maximum(cnt[...], 1.0)
        g = jnp.maximum(jnp.dot(pooled, w1[...],
                                preferred_element_type=jnp.float32)
                        + b1[...], 0.0)
        g = jnp.maximum(jnp.dot(g, w2[...],
                                preferred_element_type=jnp.float32)
                        + b2[...], 0.0)
        out_ref[...] = jnp.dot(g, w3[...],
                               preferred_element_type=jnp.float32) + b3[...]


def _node_spec(shape=(TN, H)):
    return pl.BlockSpec(shape, lambda i, c: (i, 0))


def _half_spec():
    return pl.BlockSpec((TN, HF), lambda i, c: (c * GI + i, 0))


def _halves_specs():
    a = pl.BlockSpec((TN, HF), lambda i: (i, 0))
    b = pl.BlockSpec((TN, HF), lambda i: (GI + i, 0))
    return a, b


_pre0 = pl.pallas_call(
    _pre0_body,
    grid=(GI, NC),
    in_specs=[
        pl.BlockSpec((TN, 1), lambda i, c: (i, 0)),
        pl.BlockSpec((TN, H), lambda i, c: (i, 0)),
        pl.BlockSpec((H, HF), lambda i, c: (0, c)),
    ],
    out_specs=_half_spec(),
    out_shape=jax.ShapeDtypeStruct((NC * NP, HF), jnp.float32),
    name="tc_pre0",
)


def _make_mid(residual):
    body = _mid_body_res if residual else _mid_body_nores
    sa2 = pl.BlockSpec((TN, HF), lambda i, c: (i, 0))
    sb2 = pl.BlockSpec((TN, HF), lambda i, c: (GI + i, 0))
    row = pl.BlockSpec((1, H), lambda i, c: (0, 0))
    specs = [
        pl.BlockSpec((TN, 1), lambda i, c: (i, 0)),
        sa2, sb2,
        pl.BlockSpec((TN, HF), lambda i, c: (i, 0)),
        pl.BlockSpec((TN, HF), lambda i, c: (GI + i, 0)),
        row, row, row,
    ]
    if residual:
        specs.append(pl.BlockSpec((TN, H), lambda i, c: (i, 0)))
    specs.append(pl.BlockSpec((H, HF), lambda i, c: (0, c)))
    return pl.pallas_call(
        body,
        grid=(GI, NC),
        in_specs=specs,
        out_specs=[
            pl.BlockSpec((TN, H), lambda i, c: (i, 0)),
            _half_spec(),
        ],
        out_shape=[
            jax.ShapeDtypeStruct((NP, H), jnp.float32),
            jax.ShapeDtypeStruct((NC * NP, HF), jnp.float32),
        ],
        name="tc_mid_res" if residual else "tc_mid",
    )


_mid0 = _make_mid(False)
_mid1 = _make_mid(True)

_sa1, _sb1 = _halves_specs()
_row1 = pl.BlockSpec((1, H), lambda i: (0, 0))
_fin = pl.pallas_call(
    _fin_body,
    grid=(GI,),
    in_specs=[
        pl.BlockSpec((TN, 1), lambda i: (i, 0)),
        _sa1, _sb1,
        pl.BlockSpec((TN, HF), lambda i: (i, 0)),
        pl.BlockSpec((TN, HF), lambda i: (GI + i, 0)),
        _row1, _row1, _row1,
        pl.BlockSpec((TN, H), lambda i: (i, 0)),
        pl.BlockSpec((TN, 1), lambda i: (i, 0)),
        pl.BlockSpec((H, H), lambda i: (0, 0)),
        pl.BlockSpec((1, H), lambda i: (0, 0)),
        pl.BlockSpec((H, HF), lambda i: (0, 0)),
        pl.BlockSpec((1, HF), lambda i: (0, 0)),
        pl.BlockSpec((HF, OUT), lambda i: (0, 0)),
        pl.BlockSpec((1, OUT), lambda i: (0, 0)),
    ],
    out_specs=pl.BlockSpec((NB, OUT), lambda i: (0, 0)),
    out_shape=jax.ShapeDtypeStruct((NB, OUT), jnp.float32),
    scratch_shapes=[
        pltpu.VMEM((NB, H), jnp.float32),
        pltpu.VMEM((NB, 1), jnp.float32),
    ],
    name="tc_final",
)


def kernel(x, edge_index, batch,
           conv_W0, conv_b0, conv_W1, conv_b1, conv_W2, conv_b2,
           ln_s0, ln_b0, ln_s1, ln_b1, ln_s2, ln_b2,
           gh_W1, gh_b1, gh_W2, gh_b2, gh_W3, gh_b3):
    src = edge_index[0]
    dst = edge_index[1]
    batch_p = batch.reshape(N, 1)

    deg = _deg_sc(dst).reshape(NP, 1)
    u0 = _pre0(deg, x, conv_W0)
    s0 = _edge_sc(u0, src, dst)
    h0, u1 = _mid0(deg, s0, s0, u0, u0,
                   conv_b0.reshape(1, H), ln_s0.reshape(1, H),
                   ln_b0.reshape(1, H), conv_W1)
    s1 = _edge_sc(u1, src, dst)
    h1, u2 = _mid1(deg, s1, s1, u1, u1,
                   conv_b1.reshape(1, H), ln_s1.reshape(1, H),
                   ln_b1.reshape(1, H), h0, conv_W2)
    s2 = _edge_sc(u2, src, dst)
    out = _fin(deg, s2, s2, u2, u2,
               conv_b2.reshape(1, H), ln_s2.reshape(1, H),
               ln_b2.reshape(1, H), h1, batch_p,
               gh_W1, gh_b1.reshape(1, H), gh_W2, gh_b2.reshape(1, HF),
               gh_W3, gh_b3.reshape(1, OUT))
    return out
